# baseline (speedup 1.0000x reference)
import functools

import jax
import jax.numpy as jnp
import numpy as np
from jax import lax
from jax.experimental import pallas as pl
from jax.experimental.pallas import tpu as pltpu

N = 4096
K = 1024
D = 32
TOPK = 8
BN = 8
CH = 128
NCH = K // CH

_ZVALS = tuple(float(np.float32(1.0) / np.float32(r + 1)) for r in range(TOPK))


def _body(x_ref, c_ref, ct_ref, d_ref, isort_ref, k_ref, z_ref, xc_ref,
          dt_s, rt_s, oh_s):
    x = x_ref[...]
    ct = ct_ref[...]
    x2 = jnp.sum(x * x, axis=1, keepdims=True)
    c2 = jnp.sum(ct * ct, axis=0, keepdims=True)
    xc = lax.dot_general(x, ct, (((1,), (0,)), ((), ())),
                         precision=lax.Precision.HIGHEST,
                         preferred_element_type=jnp.float32)
    d = jnp.sqrt(jnp.maximum(x2 + c2 - 2.0 * xc, 0.0))
    d_ref[...] = d
    dt_s[...] = d.T

    for t in range(BN):
        drow = d_ref[pl.ds(t, 1), :]

        def rank_chunk(ci, acc, _t=t, _drow=drow):
            dcol = dt_s[pl.ds(ci * CH, CH), pl.ds(_t, 1)]
            lt = dcol < _drow
            eq = dcol == _drow
            ii = lax.broadcasted_iota(jnp.int32, (CH, K), 0) + ci * CH
            jj = lax.broadcasted_iota(jnp.int32, (CH, K), 1)
            tie = eq & (ii < jj)
            cnt = jnp.where(lt | tie, 1.0, 0.0)
            return acc + jnp.sum(cnt, axis=0, keepdims=True)

        rank = lax.fori_loop(0, NCH, rank_chunk,
                             jnp.zeros((1, K), jnp.float32))
        k_ref[pl.ds(t, 1), :] = rank.astype(jnp.int32)

        z = jnp.zeros((1, K), jnp.float32)
        for r in range(TOPK):
            z = jnp.where(rank == float(r), _ZVALS[r], z)
        z_ref[pl.ds(t, 1), :] = z
        oh_s[pl.ds(t, 1), :] = jnp.where(rank == 0.0, 1.0, 0.0)

    rt_s[...] = k_ref[...].T

    riota = lax.broadcasted_iota(jnp.int32, (1, K), 1)
    for t in range(BN):

        def isort_chunk(ci, acc, _t=t, _riota=riota):
            rcol = rt_s[pl.ds(ci * CH, CH), pl.ds(_t, 1)]
            jcol = (lax.broadcasted_iota(jnp.int32, (CH, 1), 0)
                    + ci * CH).astype(jnp.float32)
            cnt = jnp.where(rcol == _riota, jcol, 0.0)
            return acc + jnp.sum(cnt, axis=0, keepdims=True)

        isr = lax.fori_loop(0, NCH, isort_chunk,
                            jnp.zeros((1, K), jnp.float32))
        isort_ref[pl.ds(t, 1), :] = isr.astype(jnp.int32)

    xc_ref[...] = lax.dot_general(oh_s[...], c_ref[...],
                                  (((1,), (0,)), ((), ())),
                                  precision=lax.Precision.HIGHEST,
                                  preferred_element_type=jnp.float32)


@jax.jit
def kernel(x, c):
    ct = c.T
    out = pl.pallas_call(
        _body,
        grid=(N // BN,),
        in_specs=[
            pl.BlockSpec((BN, D), lambda i: (i, 0)),
            pl.BlockSpec((K, D), lambda i: (0, 0)),
            pl.BlockSpec((D, K), lambda i: (0, 0)),
        ],
        out_specs=[
            pl.BlockSpec((BN, K), lambda i: (i, 0)),
            pl.BlockSpec((BN, K), lambda i: (i, 0)),
            pl.BlockSpec((BN, K), lambda i: (i, 0)),
            pl.BlockSpec((BN, K), lambda i: (i, 0)),
            pl.BlockSpec((BN, D), lambda i: (i, 0)),
        ],
        out_shape=[
            jax.ShapeDtypeStruct((N, K), jnp.float32),
            jax.ShapeDtypeStruct((N, K), jnp.int32),
            jax.ShapeDtypeStruct((N, K), jnp.int32),
            jax.ShapeDtypeStruct((N, K), jnp.float32),
            jax.ShapeDtypeStruct((N, D), jnp.float32),
        ],
        scratch_shapes=[
            pltpu.VMEM((K, BN), jnp.float32),
            pltpu.VMEM((K, BN), jnp.int32),
            pltpu.VMEM((BN, K), jnp.float32),
        ],
    )(x, c, ct)
    d, isort, k, z, x_c = out
    return (d, isort, k, z, x_c)


# speedup vs baseline: 3.4510x; 3.4510x over previous
import functools

import jax
import jax.numpy as jnp
import numpy as np
from jax import lax
from jax.experimental import pallas as pl
from jax.experimental.pallas import tpu as pltpu
from jax.experimental.pallas import tpu_sc as plsc

N = 4096
K = 1024
D = 32
TOPK = 8

NW = 32
RPW = N // NW
L = 16
SEG = K // L
RADIX = 32
NPASS = 7

_ZVALS = tuple(float(np.float32(1.0) / np.float32(r + 1)) for r in range(TOPK))


def _dist_body(x_ref, ct_ref, d_ref):
    x = x_ref[...]
    ct = ct_ref[...]
    x2 = jnp.sum(x * x, axis=1, keepdims=True)
    c2 = jnp.sum(ct * ct, axis=0, keepdims=True)
    xc = lax.dot_general(x, ct, (((1,), (0,)), ((), ())),
                         precision=lax.Precision.HIGHEST,
                         preferred_element_type=jnp.float32)
    d_ref[...] = jnp.sqrt(jnp.maximum(x2 + c2 - 2.0 * xc, 0.0))


def _xc_body(k_ref, c_ref, xc_ref):
    oh = jnp.where(k_ref[...] == 0, 1.0, 0.0)
    xc_ref[...] = lax.dot_general(oh, c_ref[...], (((1,), (0,)), ((), ())),
                                  precision=lax.Precision.HIGHEST,
                                  preferred_element_type=jnp.float32)


def _sc_body(d_hbm, isort_hbm, k_hbm, z_hbm,
             dbuf, key_a, key_b, pay_a, pay_b, offs, sincl, smem_s,
             kbuf, zbuf):
    wid = lax.axis_index("s") * 2 + lax.axis_index("c")
    row0 = wid * RPW
    lane = lax.iota(jnp.int32, L)
    seg_base = lane * SEG
    zero16 = jnp.zeros((L,), jnp.int32)
    one16 = jnp.ones((L,), jnp.int32)
    zvals = jnp.zeros((L,), jnp.float32)
    for r in range(TOPK):
        zvals = jnp.where(lane == r, _ZVALS[r], zvals)

    def row_body(ri, _):
        row = row0 + ri
        pltpu.sync_copy(d_hbm.at[row], dbuf)
        del _

        def one_pass(shift, src_key, src_pay, dst_key, dst_pay, first):
            def hist_step(s, _):
                idx = seg_base + s
                if first:
                    kv = plsc.bitcast(plsc.load_gather(dbuf, [idx]), jnp.int32)
                else:
                    kv = plsc.load_gather(src_key, [idx])
                digit = lax.shift_right_logical(kv, shift) & (RADIX - 1)
                hidx = digit * L + lane
                plsc.addupdate_scatter(offs, [hidx], one16)
                return 0

            def zero_step(b, _):
                offs[pl.ds(b * L, L)] = zero16
                return 0

            lax.fori_loop(0, RADIX, zero_step, 0)
            lax.fori_loop(0, SEG, hist_step, 0)

            def scan_step(b, _):
                v = offs[pl.ds(b * L, L)]
                cs = plsc.cumsum(v)
                sincl[pl.ds(b * L, L)] = cs
                offs[pl.ds(b * L, L)] = cs - v
                return 0

            lax.fori_loop(0, RADIX, scan_step, 0)
            t0 = plsc.load_gather(sincl, [lane * L + (L - 1)])
            t1 = plsc.load_gather(sincl, [RADIX * L // 2 + lane * L + (L - 1)])
            cs0 = plsc.cumsum(t0)
            carry0 = jnp.sum(t0)
            smem_s[pl.ds(0, L)] = cs0 - t0
            smem_s[pl.ds(L, L)] = plsc.cumsum(t1) - t1 + carry0

            def perm_step(s, _):
                idx = seg_base + s
                if first:
                    kv = plsc.bitcast(plsc.load_gather(dbuf, [idx]), jnp.int32)
                    pv = idx
                else:
                    kv = plsc.load_gather(src_key, [idx])
                    pv = plsc.load_gather(src_pay, [idx])
                digit = lax.shift_right_logical(kv, shift) & (RADIX - 1)
                hidx = digit * L + lane
                o1 = plsc.load_gather(offs, [hidx])
                plsc.store_scatter(offs, [hidx], o1 + 1)
                pos = o1 + plsc.load_gather(smem_s, [digit])
                plsc.store_scatter(dst_key, [pos], kv)
                plsc.store_scatter(dst_pay, [pos], pv)
                return 0

            lax.fori_loop(0, SEG, perm_step, 0)

        bufs = [key_a, key_b]
        pays = [pay_a, pay_b]
        one_pass(0, None, None, key_a, pay_a, True)
        for p in range(1, NPASS):
            one_pass(5 * p, bufs[(p + 1) % 2], pays[(p + 1) % 2],
                     bufs[p % 2], pays[p % 2], False)
        fin_pay = pays[(NPASS - 1) % 2]

        def out_step(s, _):
            pv = fin_pay[pl.ds(s * L, L)]
            plsc.store_scatter(kbuf, [pv], s * L + lane)
            zv = jnp.where(s == 0, zvals, jnp.zeros((L,), jnp.float32))
            plsc.store_scatter(zbuf, [pv], zv)
            return 0

        lax.fori_loop(0, SEG, out_step, 0)
        pltpu.sync_copy(fin_pay, isort_hbm.at[row])
        pltpu.sync_copy(kbuf, k_hbm.at[row])
        pltpu.sync_copy(zbuf, z_hbm.at[row])
        return 0

    lax.fori_loop(0, RPW, row_body, 0)


@jax.jit
def kernel(x, c):
    ct = c.T
    d = pl.pallas_call(
        _dist_body,
        grid=(N // 256,),
        in_specs=[
            pl.BlockSpec((256, D), lambda i: (i, 0)),
            pl.BlockSpec((D, K), lambda i: (0, 0)),
        ],
        out_specs=pl.BlockSpec((256, K), lambda i: (i, 0)),
        out_shape=jax.ShapeDtypeStruct((N, K), jnp.float32),
    )(x, ct)

    sc_fn = functools.partial(
        pl.kernel,
        mesh=plsc.VectorSubcoreMesh(core_axis_name="c", subcore_axis_name="s"),
        out_type=[
            jax.ShapeDtypeStruct((N, K), jnp.int32),
            jax.ShapeDtypeStruct((N, K), jnp.int32),
            jax.ShapeDtypeStruct((N, K), jnp.float32),
        ],
        scratch_types=[
            pltpu.VMEM((K,), jnp.float32),
            pltpu.VMEM((K,), jnp.int32),
            pltpu.VMEM((K,), jnp.int32),
            pltpu.VMEM((K,), jnp.int32),
            pltpu.VMEM((K,), jnp.int32),
            pltpu.VMEM((RADIX * L,), jnp.int32),
            pltpu.VMEM((RADIX * L,), jnp.int32),
            pltpu.VMEM((RADIX,), jnp.int32),
            pltpu.VMEM((K,), jnp.int32),
            pltpu.VMEM((K,), jnp.float32),
        ],
        compiler_params=pltpu.CompilerParams(needs_layout_passes=False),
    )(_sc_body)
    isort, k, z = sc_fn(d)

    x_c = pl.pallas_call(
        _xc_body,
        grid=(N // 256,),
        in_specs=[
            pl.BlockSpec((256, K), lambda i: (i, 0)),
            pl.BlockSpec((K, D), lambda i: (0, 0)),
        ],
        out_specs=pl.BlockSpec((256, D), lambda i: (i, 0)),
        out_shape=jax.ShapeDtypeStruct((N, D), jnp.float32),
    )(k, c)
    return (d, isort, k, z, x_c)
